# baseline (speedup 1.0000x reference)
import jax
import jax.numpy as jnp
from jax.experimental import pallas as pl
from jax.experimental.pallas import tpu as pltpu

_BM = 400
_HM = _BM // 2


def _half_copy(l_hbm, buf, sem, step, h):
    return pltpu.make_async_copy(
        l_hbm.at[pl.ds(step * _BM + h * _HM, _HM), :],
        buf,
        sem,
    )


def _body(l_hbm, x_ref, scale_ref, bias_ref, o_ref,
          xbf_ref, b00, b01, b10, b11, sems):
    i = pl.program_id(0)
    nsteps = pl.num_programs(0)

    @pl.when(i == 0)
    def _():
        xbf_ref[...] = x_ref[...].astype(jnp.bfloat16)
        _half_copy(l_hbm, b00, sems.at[0, 0], 0, 0).start()
        _half_copy(l_hbm, b01, sems.at[0, 1], 0, 1).start()
        _half_copy(l_hbm, b10, sems.at[1, 0], 1, 0).start()
        _half_copy(l_hbm, b11, sems.at[1, 1], 1, 1).start()

    def compute(ba, bb, sa, sb):
        _half_copy(l_hbm, ba, sa, i, 0).wait()
        e1a = jnp.dot(ba[...].astype(jnp.bfloat16), xbf_ref[...],
                      preferred_element_type=jnp.float32)
        o_ref[pl.ds(0, _HM), :] = (
            x_ref[pl.ds(i * _BM, _HM), :]
            - e1a * scale_ref[...] + bias_ref[...])
        _half_copy(l_hbm, bb, sb, i, 1).wait()
        e1b = jnp.dot(bb[...].astype(jnp.bfloat16), xbf_ref[...],
                      preferred_element_type=jnp.float32)
        o_ref[pl.ds(_HM, _HM), :] = (
            x_ref[pl.ds(i * _BM + _HM, _HM), :]
            - e1b * scale_ref[...] + bias_ref[...])

        @pl.when(i + 2 < nsteps)
        def _():
            _half_copy(l_hbm, ba, sa, i + 2, 0).start()
            _half_copy(l_hbm, bb, sb, i + 2, 1).start()

    @pl.when(jax.lax.rem(i, 2) == 0)
    def _():
        compute(b00, b01, sems.at[0, 0], sems.at[0, 1])

    @pl.when(jax.lax.rem(i, 2) == 1)
    def _():
        compute(b10, b11, sems.at[1, 0], sems.at[1, 1])


def kernel(input, l_sym, learnable_diag_1, bias):
    n, f = input.shape
    scale2d = learnable_diag_1.reshape(1, f)
    bias2d = bias.reshape(1, f)
    return pl.pallas_call(
        _body,
        grid=(n // _BM,),
        in_specs=[
            pl.BlockSpec(memory_space=pltpu.MemorySpace.HBM),
            pl.BlockSpec((n, f), lambda i: (0, 0)),
            pl.BlockSpec((1, f), lambda i: (0, 0)),
            pl.BlockSpec((1, f), lambda i: (0, 0)),
        ],
        out_specs=pl.BlockSpec((_BM, f), lambda i: (i, 0)),
        out_shape=jax.ShapeDtypeStruct((n, f), jnp.float32),
        scratch_shapes=[
            pltpu.VMEM((n, f), jnp.bfloat16),
            pltpu.VMEM((_HM, n), jnp.float32),
            pltpu.VMEM((_HM, n), jnp.float32),
            pltpu.VMEM((_HM, n), jnp.float32),
            pltpu.VMEM((_HM, n), jnp.float32),
            pltpu.SemaphoreType.DMA((2, 2)),
        ],
    )(l_sym, input, scale2d, bias2d)


# speedup vs baseline: 1.0110x; 1.0110x over previous
import jax
import jax.numpy as jnp
from jax.experimental import pallas as pl
from jax.experimental.pallas import tpu as pltpu

_BM = 400


def _body(l_ref, x_ref, scale_ref, bias_ref, o_ref, xbf_ref):
    i = pl.program_id(0)

    @pl.when(i == 0)
    def _():
        xbf_ref[...] = x_ref[...].astype(jnp.bfloat16)

    e1 = jnp.dot(
        l_ref[...].astype(jnp.bfloat16),
        xbf_ref[...],
        preferred_element_type=jnp.float32,
    )
    rows = x_ref[pl.ds(i * _BM, _BM), :]
    o_ref[...] = rows - e1 * scale_ref[...] + bias_ref[...]


def kernel(input, l_sym, learnable_diag_1, bias):
    n, f = input.shape
    scale2d = learnable_diag_1.reshape(1, f)
    bias2d = bias.reshape(1, f)
    return pl.pallas_call(
        _body,
        grid=(n // _BM,),
        in_specs=[
            pl.BlockSpec((_BM, n), lambda i: (i, 0)),
            pl.BlockSpec((n, f), lambda i: (0, 0)),
            pl.BlockSpec((1, f), lambda i: (0, 0)),
            pl.BlockSpec((1, f), lambda i: (0, 0)),
        ],
        out_specs=pl.BlockSpec((_BM, f), lambda i: (i, 0)),
        out_shape=jax.ShapeDtypeStruct((n, f), jnp.float32),
        scratch_shapes=[pltpu.VMEM((n, f), jnp.bfloat16)],
    )(l_sym, input, scale2d, bias2d)
